# baseline (speedup 1.0000x reference)
import functools

import jax
import jax.numpy as jnp
from jax import lax
from jax.experimental import pallas as pl
from jax.experimental.pallas import tpu as pltpu
from jax.experimental.pallas import tpu_sc as plsc

_NC = 2
_NS = 16
_NW = _NC * _NS
_C = 8
_RING = 3


def _sc_body(x_hbm, pe_hbm, out_hbm, pe_v, x_v, sem_pe, sem_ld, sem_st,
             *, B, S, D):
    wid = lax.axis_index("s") * _NC + lax.axis_index("c")
    ppw = S // _NW
    nch = ppw // _C
    base = wid * ppw

    def start_pe(g, sl):
        pltpu.async_copy(
            pe_hbm.at[pl.ds(base + g * _C, _C)], pe_v.at[sl], sem_pe.at[sl])

    def wait_pe(sl):
        pltpu.make_async_copy(
            pe_hbm.at[pl.ds(base, _C)], pe_v.at[sl], sem_pe.at[sl]).wait()

    def start_ld(g, sl):
        pltpu.async_copy(
            x_hbm.at[:, pl.ds(base + g * _C, _C)], x_v.at[sl], sem_ld.at[sl])

    def wait_ld(sl):
        pltpu.make_async_copy(
            x_hbm.at[:, pl.ds(base, _C)], x_v.at[sl], sem_ld.at[sl]).wait()

    def start_st(g, sl, b):
        pltpu.async_copy(
            x_v.at[sl, b], out_hbm.at[b, pl.ds(base + g * _C, _C)],
            sem_st.at[sl])

    def wait_st(sl, b):
        pltpu.make_async_copy(
            x_v.at[sl, b], out_hbm.at[b, pl.ds(base, _C)],
            sem_st.at[sl]).wait()

    def run_chunk(g, sl, prefetch_g, drain=True):
        wait_pe(sl)
        wait_ld(sl)
        for b in range(B):
            @plsc.parallel_loop(0, D, step=16, unroll=2)
            def _(i):
                for rw in range(_C):
                    plsc.addupdate(x_v.at[sl, b, rw, pl.ds(i, 16)],
                                   pe_v.at[sl][rw, pl.ds(i, 16)])

            start_st(g, sl, b)
        if prefetch_g is not None:
            sl2 = (sl + 2) % _RING
            if drain:
                for b in range(B):
                    wait_st(sl2, b)
            start_ld(prefetch_g, sl2)
            start_pe(prefetch_g, sl2)

    start_pe(0, 0)
    start_ld(0, 0)
    start_pe(1, 1)
    start_ld(1, 1)

    run_chunk(0, 0, 2, drain=False)
    run_chunk(1, 1, 3)
    run_chunk(2, 2, 4)

    n_groups = (nch - 5) // _RING

    def group_body(gi, _):
        g0 = 3 + gi * _RING
        for j in range(_RING):
            run_chunk(g0 + j, j, g0 + j + 2)
        return ()

    lax.fori_loop(0, n_groups, group_body, ())

    run_chunk(nch - 2, (nch - 2) % _RING, None)
    run_chunk(nch - 1, (nch - 1) % _RING, None)

    for g in range(nch - 3, nch):
        for b in range(B):
            wait_st(g % _RING, b)


def kernel(x, pe):
    B, S, D = x.shape

    mesh = plsc.VectorSubcoreMesh(core_axis_name="c", subcore_axis_name="s")
    k = pl.kernel(
        functools.partial(_sc_body, B=B, S=S, D=D),
        out_type=jax.ShapeDtypeStruct((B, S, D), jnp.float32),
        mesh=mesh,
        scratch_types=[
            pltpu.VMEM((_RING, _C, D), jnp.float32),
            pltpu.VMEM((_RING, B, _C, D), jnp.float32),
            pltpu.SemaphoreType.DMA((_RING,)),
            pltpu.SemaphoreType.DMA((_RING,)),
            pltpu.SemaphoreType.DMA((_RING,)),
        ],
    )
    return k(x, pe[:S])


# speedup vs baseline: 1.0091x; 1.0091x over previous
import functools

import jax
import jax.numpy as jnp
from jax import lax
from jax.experimental import pallas as pl
from jax.experimental.pallas import tpu as pltpu
from jax.experimental.pallas import tpu_sc as plsc

_NC = 2
_NS = 16
_NW = _NC * _NS
_C = 16
_PR = 2
_LEAD = 2


def _sc_body(x_hbm, pe_hbm, out_hbm, pe_v, x_v, sem_pe, sem_ld, sem_st,
             *, B, S, D):
    wid = lax.axis_index("s") * _NC + lax.axis_index("c")
    ppw = S // _NW
    nch = ppw // _C
    ntask = nch * B
    base = wid * ppw

    def start_pe(g, sl):
        pltpu.async_copy(
            pe_hbm.at[pl.ds(base + g * _C, _C)], pe_v.at[sl], sem_pe.at[sl])

    def wait_pe(sl):
        pltpu.make_async_copy(
            pe_hbm.at[pl.ds(base, _C)], pe_v.at[sl], sem_pe.at[sl]).wait()

    def start_ld(g, b):
        pltpu.async_copy(
            x_hbm.at[b, pl.ds(base + g * _C, _C)], x_v.at[b], sem_ld.at[b])

    def wait_ld(b):
        pltpu.make_async_copy(
            x_hbm.at[b, pl.ds(base, _C)], x_v.at[b], sem_ld.at[b]).wait()

    def start_st(g, b):
        pltpu.async_copy(
            x_v.at[b], out_hbm.at[b, pl.ds(base + g * _C, _C)], sem_st.at[b])

    def wait_st(b):
        pltpu.make_async_copy(
            x_v.at[b], out_hbm.at[b, pl.ds(base, _C)], sem_st.at[b]).wait()

    def run_task(g, b, psl, *, pe_next=None, drain=True, load_next=True):
        if pe_next is not None:
            start_pe(g + 1, pe_next)
        if b == 0:
            wait_pe(psl)
        if load_next:
            b2 = (b + _LEAD) % B
            g2 = g + (b + _LEAD) // B
            if drain:
                wait_st(b2)
            start_ld(g2, b2)
        wait_ld(b)

        @plsc.parallel_loop(0, D, step=16, unroll=1)
        def _(i):
            for rw in range(_C):
                plsc.addupdate(x_v.at[b, rw, pl.ds(i, 16)],
                               pe_v.at[psl][rw, pl.ds(i, 16)])

        start_st(g, b)

    start_pe(0, 0)
    start_ld(0, 0)
    start_ld(0, 1)

    run_task(0, 0, 0, pe_next=1, drain=False)
    run_task(0, 1, 0, drain=False)
    run_task(0, 2, 0)
    run_task(0, 3, 0)
    run_task(1, 0, 1, pe_next=0)
    run_task(1, 1, 1)
    run_task(1, 2, 1)
    run_task(1, 3, 1)

    n_groups = (nch - 4) // _PR

    def group_body(gi, _):
        g0 = 2 + gi * _PR
        run_task(g0, 0, 0, pe_next=1)
        run_task(g0, 1, 0)
        run_task(g0, 2, 0)
        run_task(g0, 3, 0)
        run_task(g0 + 1, 0, 1, pe_next=0)
        run_task(g0 + 1, 1, 1)
        run_task(g0 + 1, 2, 1)
        run_task(g0 + 1, 3, 1)
        return ()

    lax.fori_loop(0, n_groups, group_body, ())

    gt = nch - 2
    run_task(gt, 0, gt % _PR, pe_next=(gt + 1) % _PR)
    run_task(gt, 1, gt % _PR)
    run_task(gt, 2, gt % _PR)
    run_task(gt, 3, gt % _PR)
    gt = nch - 1
    run_task(gt, 0, gt % _PR)
    run_task(gt, 1, gt % _PR)
    run_task(gt, 2, gt % _PR, load_next=False)
    run_task(gt, 3, gt % _PR, load_next=False)

    for b in range(B - _LEAD, B):
        wait_st(b)


def kernel(x, pe):
    B, S, D = x.shape

    mesh = plsc.VectorSubcoreMesh(core_axis_name="c", subcore_axis_name="s")
    k = pl.kernel(
        functools.partial(_sc_body, B=B, S=S, D=D),
        out_type=jax.ShapeDtypeStruct((B, S, D), jnp.float32),
        mesh=mesh,
        scratch_types=[
            pltpu.VMEM((_PR, _C, D), jnp.float32),
            pltpu.VMEM((B, _C, D), jnp.float32),
            pltpu.SemaphoreType.DMA((_PR,)),
            pltpu.SemaphoreType.DMA((B,)),
            pltpu.SemaphoreType.DMA((B,)),
        ],
    )
    return k(x, pe[:S])


# speedup vs baseline: 1.0194x; 1.0102x over previous
import functools

import jax
import jax.numpy as jnp
from jax import lax
from jax.experimental import pallas as pl
from jax.experimental.pallas import tpu as pltpu
from jax.experimental.pallas import tpu_sc as plsc

_NC = 2
_NS = 16
_NW = _NC * _NS
_C = 8
_RING = 3


def _sc_body(x_hbm, pe_hbm, out_hbm, pe_v, x_v, sem_pe, sem_ld, sem_st,
             *, B, S, D):
    wid = lax.axis_index("s") * _NC + lax.axis_index("c")
    ppw = S // _NW
    nch = ppw // _C
    base = wid * ppw

    def start_pe(g, sl):
        pltpu.async_copy(
            pe_hbm.at[pl.ds(base + g * _C, _C)], pe_v.at[sl], sem_pe.at[sl])

    def wait_pe(sl):
        pltpu.make_async_copy(
            pe_hbm.at[pl.ds(base, _C)], pe_v.at[sl], sem_pe.at[sl]).wait()

    def start_ld(g, sl):
        pltpu.async_copy(
            x_hbm.at[:, pl.ds(base + g * _C, _C)], x_v.at[sl], sem_ld.at[sl])

    def wait_ld(sl):
        pltpu.make_async_copy(
            x_hbm.at[:, pl.ds(base, _C)], x_v.at[sl], sem_ld.at[sl]).wait()

    def start_st(g, sl, b):
        pltpu.async_copy(
            x_v.at[sl, b], out_hbm.at[b, pl.ds(base + g * _C, _C)],
            sem_st.at[sl])

    def wait_st(sl, b):
        pltpu.make_async_copy(
            x_v.at[sl, b], out_hbm.at[b, pl.ds(base, _C)],
            sem_st.at[sl]).wait()

    def run_chunk(g, sl, prefetch_g, drain=True):
        wait_pe(sl)
        wait_ld(sl)
        for b in range(B):
            @plsc.parallel_loop(0, D, step=16, unroll=1)
            def _(i):
                for rw in range(_C):
                    plsc.addupdate(x_v.at[sl, b, rw, pl.ds(i, 16)],
                                   pe_v.at[sl][rw, pl.ds(i, 16)])

            start_st(g, sl, b)
        if prefetch_g is not None:
            sl2 = (sl + 2) % _RING
            if drain:
                for b in range(B):
                    wait_st(sl2, b)
            start_ld(prefetch_g, sl2)
            start_pe(prefetch_g, sl2)

    start_pe(0, 0)
    start_ld(0, 0)
    start_pe(1, 1)
    start_ld(1, 1)

    run_chunk(0, 0, 2, drain=False)
    run_chunk(1, 1, 3)
    run_chunk(2, 2, 4)

    n_groups = (nch - 5) // _RING

    def group_body(gi, _):
        g0 = 3 + gi * _RING
        for j in range(_RING):
            run_chunk(g0 + j, j, g0 + j + 2)
        return ()

    lax.fori_loop(0, n_groups, group_body, ())

    run_chunk(nch - 2, (nch - 2) % _RING, None)
    run_chunk(nch - 1, (nch - 1) % _RING, None)

    for g in range(nch - 3, nch):
        for b in range(B):
            wait_st(g % _RING, b)


def kernel(x, pe):
    B, S, D = x.shape

    mesh = plsc.VectorSubcoreMesh(core_axis_name="c", subcore_axis_name="s")
    k = pl.kernel(
        functools.partial(_sc_body, B=B, S=S, D=D),
        out_type=jax.ShapeDtypeStruct((B, S, D), jnp.float32),
        mesh=mesh,
        scratch_types=[
            pltpu.VMEM((_RING, _C, D), jnp.float32),
            pltpu.VMEM((_RING, B, _C, D), jnp.float32),
            pltpu.SemaphoreType.DMA((_RING,)),
            pltpu.SemaphoreType.DMA((_RING,)),
            pltpu.SemaphoreType.DMA((_RING,)),
        ],
    )
    return k(x, pe[:S])
